# baseline (speedup 1.0000x reference)
from functools import partial

import jax
import jax.numpy as jnp
from jax.experimental import pallas as pl
from jax.experimental.pallas import tpu as pltpu


def _sign(x):
    return jnp.where(x > 0, 1.0, jnp.where(x < 0, -1.0, 0.0))


def _fused_kernel(x_ref, w2t_ref, drop_ref, w3r_ref, w0_ref,
                  o_ref, b_ref, inv2_ref, bacc_ref, *, n_i, kk, n_o):
    i = pl.program_id(0)

    @pl.when(i == 0)
    def _():
        w2t = w2t_ref[...]
        inv2_ref[...] = jax.lax.rsqrt(
            jnp.sum(w2t * w2t, axis=0, keepdims=True) + 1e-6)

    e = x_ref.shape[1]
    b = jnp.zeros((1, w0_ref.shape[0]), jnp.float32)
    for j in range(n_i):
        b = b + jax.lax.dot_general(
            x_ref[j:j + 1, :], w0_ref[:, j * e:(j + 1) * e],
            (((1,), (1,)), ((), ())), preferred_element_type=jnp.float32)
    bacc_ref[i // 2, i % 2, :] = _sign(b)[0]

    @pl.when(i == n_o - 1)
    def _():
        b_ref[...] = jnp.concatenate(
            [bacc_ref[:, 0, :], bacc_ref[:, 1, :]], axis=1)

    h16 = jnp.dot(x_ref[...], w2t_ref[...], preferred_element_type=jnp.float32)
    h16 = h16 * inv2_ref[...] * drop_ref[...]

    cout = h16.shape[1] // e
    hm = jnp.stack([h16[:, c * e:(c + 1) * e] for c in range(cout)],
                   axis=1).reshape(n_i * cout, e)

    c3 = w3r_ref.shape[1]
    cin = c3 // kk
    p = _sign(jnp.dot(hm, w3r_ref[...], preferred_element_type=jnp.float32))

    for q in range(kk):
        o_ref[q] = jnp.concatenate(
            [p[j * cout:(j + 1) * cout, q * cin:(q + 1) * cin]
             for j in range(n_i)], axis=1)


def kernel(embed, w0, w2, w3, drop_scale):
    N_O, N_I, E = embed.shape
    Cout = w0.shape[0]
    C3 = w3.shape[0]
    C2 = w2.shape[0]
    N = N_O * N_I
    K = 3
    KK = K * K
    Cin = C3 // KK
    f32 = jnp.float32

    x = embed.reshape(N, E).astype(f32)
    w2_t = jnp.transpose(w2).astype(f32)
    w3_r = jnp.transpose(w3).astype(f32).reshape(E, Cin, KK)
    w3_r = jnp.transpose(w3_r, (0, 2, 1)).reshape(E, C3)

    o9, b2 = pl.pallas_call(
        partial(_fused_kernel, n_i=N_I, kk=KK, n_o=N_O),
        out_shape=(jax.ShapeDtypeStruct((KK, N_O * Cout, N_I * Cin), f32),
                   jax.ShapeDtypeStruct((N_O // 2, 2 * Cout), f32)),
        grid=(N_O,),
        in_specs=[
            pl.BlockSpec((N_I, E), lambda i: (i, 0)),
            pl.BlockSpec((E, C2), lambda i: (0, 0)),
            pl.BlockSpec((N_I, C2), lambda i: (i, 0)),
            pl.BlockSpec((E, C3), lambda i: (0, 0)),
            pl.BlockSpec((Cout, N_I * E), lambda i: (0, 0)),
        ],
        out_specs=(pl.BlockSpec((KK, Cout, N_I * Cin), lambda i: (0, i, 0)),
                   pl.BlockSpec((N_O // 2, 2 * Cout), lambda i: (0, 0))),
        scratch_shapes=[pltpu.VMEM((1, C2), f32),
                        pltpu.VMEM((N_O // 2, 2, Cout), f32)],
        compiler_params=pltpu.CompilerParams(
            dimension_semantics=("arbitrary",)),
    )(x, w2_t, drop_scale, w3_r, w0.astype(f32))

    weight = jnp.transpose(o9.reshape(K, K, N_O * Cout, N_I * Cin),
                           (2, 3, 0, 1))
    bias = b2.reshape(N_O * Cout, 1)
    return weight, bias


# speedup vs baseline: 1.2081x; 1.2081x over previous
from functools import partial

import jax
import jax.numpy as jnp
from jax.experimental import pallas as pl
from jax.experimental.pallas import tpu as pltpu


def _sign(x):
    return jnp.where(x > 0, 1.0, jnp.where(x < 0, -1.0, 0.0))


def _inv_norm_kernel(w2t_ref, inv_ref):
    w2t = w2t_ref[...]
    inv_ref[...] = jax.lax.rsqrt(
        jnp.sum(w2t * w2t, axis=0, keepdims=True) + 1e-6)


def _fused_kernel(x_ref, w2t_ref, inv2_ref, drop_ref, w3r_ref, w0_ref,
                  o_ref, b_ref, *, n_i, kk):
    e = x_ref.shape[1]
    b = jnp.zeros((1, w0_ref.shape[0]), jnp.float32)
    for j in range(n_i):
        b = b + jax.lax.dot_general(
            x_ref[j:j + 1, :], w0_ref[:, j * e:(j + 1) * e],
            (((1,), (1,)), ((), ())), preferred_element_type=jnp.float32)
    b_ref[...] = _sign(b)[None]

    h16 = jnp.dot(x_ref[...], w2t_ref[...], preferred_element_type=jnp.float32)
    h16 = h16 * inv2_ref[...] * drop_ref[...]

    cout = h16.shape[1] // e
    hm = jnp.stack([h16[:, c * e:(c + 1) * e] for c in range(cout)],
                   axis=1).reshape(n_i * cout, e)

    c3 = w3r_ref.shape[1]
    cin = c3 // kk
    p = _sign(jnp.dot(hm, w3r_ref[...], preferred_element_type=jnp.float32))

    for q in range(kk):
        o_ref[q] = jnp.concatenate(
            [p[j * cout:(j + 1) * cout, q * cin:(q + 1) * cin]
             for j in range(n_i)], axis=1)


def kernel(embed, w0, w2, w3, drop_scale):
    N_O, N_I, E = embed.shape
    Cout = w0.shape[0]
    C3 = w3.shape[0]
    C2 = w2.shape[0]
    N = N_O * N_I
    K = 3
    KK = K * K
    Cin = C3 // KK
    f32 = jnp.float32

    x = embed.reshape(N, E).astype(f32)
    w2_t = jnp.transpose(w2).astype(f32)
    w3_r = jnp.transpose(w3).astype(f32).reshape(E, Cin, KK)
    w3_r = jnp.transpose(w3_r, (0, 2, 1)).reshape(E, C3)

    inv2 = pl.pallas_call(
        _inv_norm_kernel,
        out_shape=jax.ShapeDtypeStruct((1, C2), f32),
        grid=(1,),
        in_specs=[pl.BlockSpec((E, C2), lambda i: (0, 0))],
        out_specs=pl.BlockSpec((1, C2), lambda i: (0, 0)),
        compiler_params=pltpu.CompilerParams(
            dimension_semantics=("arbitrary",)),
    )(w2_t)

    o9, b = pl.pallas_call(
        partial(_fused_kernel, n_i=N_I, kk=KK),
        out_shape=(jax.ShapeDtypeStruct((KK, N_O * Cout, N_I * Cin), f32),
                   jax.ShapeDtypeStruct((N_O, 1, Cout), f32)),
        grid=(N_O,),
        in_specs=[
            pl.BlockSpec((N_I, E), lambda i: (i, 0)),
            pl.BlockSpec((E, C2), lambda i: (0, 0)),
            pl.BlockSpec((1, C2), lambda i: (0, 0)),
            pl.BlockSpec((N_I, C2), lambda i: (i, 0)),
            pl.BlockSpec((E, C3), lambda i: (0, 0)),
            pl.BlockSpec((Cout, N_I * E), lambda i: (0, 0)),
        ],
        out_specs=(pl.BlockSpec((KK, Cout, N_I * Cin), lambda i: (0, i, 0)),
                   pl.BlockSpec((1, 1, Cout), lambda i: (i, 0, 0))),
        compiler_params=pltpu.CompilerParams(
            dimension_semantics=("parallel",)),
    )(x, w2_t, inv2, drop_scale, w3_r, w0.astype(f32))

    weight = jnp.transpose(o9.reshape(K, K, N_O * Cout, N_I * Cin),
                           (2, 3, 0, 1))
    bias = b.reshape(N_O * Cout, 1)
    return weight, bias
